# Initial kernel scaffold; baseline (speedup 1.0000x reference)
import jax
import jax.numpy as jnp
from jax.experimental import pallas as pl


def kernel(x, edge_index, W1, b1, W2, b2, W3, b3):
    raise NotImplementedError("write your pallas kernel here")


# speedup vs baseline: 6.6309x; 6.6309x over previous
import functools

import jax
import jax.numpy as jnp
from jax import lax
from jax.experimental import pallas as pl
from jax.experimental.pallas import tpu as pltpu
from jax.experimental.pallas import tpu_sc as plsc

N_NODES = 10000
D_IN = 128
D_HID = 128
D_EMB = 64
N_EDGES = 320000

NP = 10240
NC = 2
NS = 16
NW = NC * NS
K = 128
CH = 80
EP = NW * CH * K
RPT = NP // NS
NPARTS = RPT // K

_mesh = plsc.VectorSubcoreMesh(core_axis_name="c", subcore_axis_name="s")


def _fill(buf, rows, value):
  @pl.loop(0, rows)
  def _(r):
    vec = jnp.full((16,), value, jnp.float32)
    for cc in range(8):
      buf[r, pl.ds(cc * 16, 16)] = vec


def _deg_body(dst_hbm, out_hbm, dstv, buf, shared):
  c = lax.axis_index("c")
  s = lax.axis_index("s")
  wid = c * NS + s

  _fill(buf, K, 0.0)
  for part in range(NPARTS):
    pltpu.sync_copy(buf, shared.at[pl.ds(s * RPT + part * K, K)])
  _fill(buf, K, 1.0)
  plsc.subcore_barrier()

  pltpu.sync_copy(dst_hbm.at[wid], dstv)

  @pl.loop(0, CH)
  def _(j):
    pltpu.sync_copy(buf, shared.at[dstv.at[j]], add=True)

  plsc.subcore_barrier()
  for part in range(NPARTS):
    pltpu.sync_copy(shared.at[pl.ds(s * RPT + part * K, K)], buf)
    pltpu.sync_copy(buf, out_hbm.at[pl.ds(c * NP + s * RPT + part * K, K)])


_deg_kernel = functools.partial(
    pl.kernel,
    out_type=jax.ShapeDtypeStruct((NC * NP, 128), jnp.float32),
    mesh=_mesh,
    scratch_types=[
        pltpu.VMEM((CH, K), jnp.int32),
        pltpu.VMEM((K, 128), jnp.float32),
        pltpu.VMEM_SHARED((NP, 128), jnp.float32),
    ],
)(_deg_body)


def _spmm_body(xp_hbm, src_hbm, dst_hbm, out_hbm, srcv, dstv, buf, shared):
  c = lax.axis_index("c")
  s = lax.axis_index("s")
  wid = c * NS + s

  _fill(buf, K, 0.0)
  for part in range(NPARTS):
    pltpu.sync_copy(buf, shared.at[pl.ds(s * RPT + part * K, K)])
  plsc.subcore_barrier()

  pltpu.sync_copy(src_hbm.at[wid], srcv)
  pltpu.sync_copy(dst_hbm.at[wid], dstv)

  @pl.loop(0, CH)
  def _(j):
    pltpu.sync_copy(xp_hbm.at[srcv.at[j]], buf)
    pltpu.sync_copy(buf, shared.at[dstv.at[j]], add=True)

  plsc.subcore_barrier()
  for part in range(NPARTS):
    pltpu.sync_copy(shared.at[pl.ds(s * RPT + part * K, K)], buf)
    pltpu.sync_copy(buf, out_hbm.at[pl.ds(c * NP + s * RPT + part * K, K)])


_spmm128 = functools.partial(
    pl.kernel,
    out_type=jax.ShapeDtypeStruct((NC * NP, D_HID), jnp.float32),
    mesh=_mesh,
    scratch_types=[
        pltpu.VMEM((CH, K), jnp.int32),
        pltpu.VMEM((CH, K), jnp.int32),
        pltpu.VMEM((K, D_HID), jnp.float32),
        pltpu.VMEM_SHARED((NP, D_HID), jnp.float32),
    ],
)(_spmm_body)


_BR = 512


def _dis_body(d0_ref, d1_ref, mask_ref, out_ref):
  deg = d0_ref[...] + d1_ref[...] + 1.0
  out_ref[...] = mask_ref[...] * lax.rsqrt(deg)


def _dis_kernel(d0, d1, mask):
  return pl.pallas_call(
      _dis_body,
      out_shape=jax.ShapeDtypeStruct((NP, 1), jnp.float32),
  )(d0, d1, mask)


def _pre_body(a_ref, w_ref, dis_ref, out_ref):
  h = jnp.dot(a_ref[...], w_ref[...], preferred_element_type=jnp.float32)
  out_ref[...] = h * dis_ref[...]


def _pre_kernel(a, w, dis):
  din, dout = w.shape
  return pl.pallas_call(
      _pre_body,
      grid=(NP // _BR,),
      in_specs=[
          pl.BlockSpec((_BR, din), lambda i: (i, 0)),
          pl.BlockSpec((din, dout), lambda i: (0, 0)),
          pl.BlockSpec((_BR, 1), lambda i: (i, 0)),
      ],
      out_specs=pl.BlockSpec((_BR, dout), lambda i: (i, 0)),
      out_shape=jax.ShapeDtypeStruct((NP, dout), jnp.float32),
  )(a, w, dis)


def _mid_body(z0_ref, z1_ref, xp_ref, dis_ref, b_ref, w_ref, out_ref):
  h = dis_ref[...] * (z0_ref[...] + z1_ref[...] + xp_ref[...]) + b_ref[...]
  a = jnp.maximum(h, 0.0)
  out_ref[...] = (
      jnp.dot(a, w_ref[...], preferred_element_type=jnp.float32)
      * dis_ref[...])


def _mid_kernel(z0, z1, xp, dis, b, w):
  din, dout = w.shape
  return pl.pallas_call(
      _mid_body,
      grid=(NP // _BR,),
      in_specs=[
          pl.BlockSpec((_BR, din), lambda i: (i, 0)),
          pl.BlockSpec((_BR, din), lambda i: (i, 0)),
          pl.BlockSpec((_BR, din), lambda i: (i, 0)),
          pl.BlockSpec((_BR, 1), lambda i: (i, 0)),
          pl.BlockSpec((1, din), lambda i: (0, 0)),
          pl.BlockSpec((din, dout), lambda i: (0, 0)),
      ],
      out_specs=pl.BlockSpec((_BR, dout), lambda i: (i, 0)),
      out_shape=jax.ShapeDtypeStruct((NP, dout), jnp.float32),
  )(z0, z1, xp, dis, b, w)


def _final_body(z0_ref, z1_ref, xp_ref, dis_ref, b_ref, out_ref):
  h = dis_ref[...] * (z0_ref[...] + z1_ref[...] + xp_ref[...]) + b_ref[...]
  nrm = jnp.sqrt(jnp.sum(h * h, axis=1, keepdims=True))
  out_ref[...] = h / jnp.maximum(nrm, 1e-12)


def _final_kernel(z0, z1, xp, dis, b):
  d = b.shape[1]
  return pl.pallas_call(
      _final_body,
      grid=(NP // _BR,),
      in_specs=[
          pl.BlockSpec((_BR, d), lambda i: (i, 0)),
          pl.BlockSpec((_BR, d), lambda i: (i, 0)),
          pl.BlockSpec((_BR, d), lambda i: (i, 0)),
          pl.BlockSpec((_BR, 1), lambda i: (i, 0)),
          pl.BlockSpec((1, d), lambda i: (0, 0)),
      ],
      out_specs=pl.BlockSpec((_BR, d), lambda i: (i, 0)),
      out_shape=jax.ShapeDtypeStruct((NP, d), jnp.float32),
  )(z0, z1, xp, dis, b)


def kernel(x, edge_index, W1, b1, W2, b2, W3, b3):
  ei = edge_index.astype(jnp.int32)
  pad = jnp.full((EP - N_EDGES,), N_NODES, jnp.int32)
  srcp = jnp.concatenate([ei[0], pad]).reshape(NW, CH, K)
  dstp = jnp.concatenate([ei[1], pad]).reshape(NW, CH, K)

  xpad = jnp.pad(x, ((0, NP - N_NODES), (0, 0)))
  mask = (jnp.arange(NP) < N_NODES).astype(jnp.float32).reshape(NP, 1)

  degp = _deg_kernel(dstp)
  dis = _dis_kernel(degp[:NP, :1], degp[NP:, :1], mask)

  xp1 = _pre_kernel(xpad, W1, dis)
  zz = _spmm128(xp1, srcp, dstp)
  xp2 = _mid_kernel(zz[:NP], zz[NP:], xp1, dis, b1.reshape(1, -1), W2)
  zz = _spmm128(xp2, srcp, dstp)
  W3p = jnp.pad(W3, ((0, 0), (0, D_HID - D_EMB)))
  xp3 = _mid_kernel(zz[:NP], zz[NP:], xp2, dis, b2.reshape(1, -1), W3p)
  zz = _spmm128(xp3, srcp, dstp)
  emb = _final_kernel(zz[:NP, :D_EMB], zz[NP:, :D_EMB], xp3[:, :D_EMB],
                      dis, b3.reshape(1, -1))
  return emb[:N_NODES]


# speedup vs baseline: 7.4143x; 1.1181x over previous
import functools

import jax
import jax.numpy as jnp
from jax import lax
from jax.experimental import pallas as pl
from jax.experimental.pallas import tpu as pltpu
from jax.experimental.pallas import tpu_sc as plsc

N_NODES = 10000
D_IN = 128
D_HID = 128
D_EMB = 64
N_EDGES = 320000

NP = 10240
NC = 2
NS = 16
NW = NC * NS
K = 128
CH = 80
EP = NW * CH * K
RPT = NP // NS
NPARTS = RPT // K

_mesh = plsc.VectorSubcoreMesh(core_axis_name="c", subcore_axis_name="s")


def _fill(buf, rows, value):
  @pl.loop(0, rows)
  def _(r):
    vec = jnp.full((16,), value, jnp.float32)
    for cc in range(8):
      buf[r, pl.ds(cc * 16, 16)] = vec


def _deg_body(dst_hbm, out_hbm, dstv, buf, shared):
  c = lax.axis_index("c")
  s = lax.axis_index("s")
  wid = c * NS + s

  _fill(buf, K, 0.0)
  for part in range(NPARTS):
    pltpu.sync_copy(buf, shared.at[pl.ds(s * RPT + part * K, K)])
  _fill(buf, K, 1.0)
  plsc.subcore_barrier()

  pltpu.sync_copy(dst_hbm.at[wid], dstv)

  @pl.loop(0, CH)
  def _(j):
    pltpu.sync_copy(buf, shared.at[dstv.at[j]], add=True)

  plsc.subcore_barrier()
  for part in range(NPARTS):
    pltpu.sync_copy(shared.at[pl.ds(s * RPT + part * K, K)], buf)
    pltpu.sync_copy(buf, out_hbm.at[pl.ds(c * NP + s * RPT + part * K, K)])


_deg_kernel = functools.partial(
    pl.kernel,
    out_type=jax.ShapeDtypeStruct((NC * NP, 128), jnp.float32),
    mesh=_mesh,
    scratch_types=[
        pltpu.VMEM((CH, K), jnp.int32),
        pltpu.VMEM((K, 128), jnp.float32),
        pltpu.VMEM_SHARED((NP, 128), jnp.float32),
    ],
)(_deg_body)


HCH = CH // 2


def _spmm_body(xp_hbm, src_hbm, dst_hbm, out_hbm, srcv, dstv, bufa, bufb,
               gsa, gsb, shared):
  c = lax.axis_index("c")
  s = lax.axis_index("s")
  wid = c * NS + s

  _fill(bufa, K, 0.0)
  for part in range(NPARTS):
    pltpu.sync_copy(bufa, shared.at[pl.ds(s * RPT + part * K, K)])
  plsc.subcore_barrier()

  for h in range(2):
    pltpu.sync_copy(src_hbm.at[wid, pl.ds(h * HCH, HCH)], srcv)
    pltpu.sync_copy(dst_hbm.at[wid, pl.ds(h * HCH, HCH)], dstv)
    pltpu.async_copy(xp_hbm.at[srcv.at[0]], bufa, gsa)

    @pl.loop(0, HCH, step=2)
    def _(jj):
      pltpu.async_copy(xp_hbm.at[srcv.at[jj + 1]], bufb, gsb)
      pltpu.make_async_copy(xp_hbm.at[srcv.at[jj]], bufa, gsa).wait()
      pltpu.sync_copy(bufa, shared.at[dstv.at[jj]], add=True)

      @pl.when(jj + 2 < HCH)
      def _():
        pltpu.async_copy(xp_hbm.at[srcv.at[jj + 2]], bufa, gsa)

      pltpu.make_async_copy(xp_hbm.at[srcv.at[jj + 1]], bufb, gsb).wait()
      pltpu.sync_copy(bufb, shared.at[dstv.at[jj + 1]], add=True)

  plsc.subcore_barrier()
  for part in range(NPARTS):
    pltpu.sync_copy(shared.at[pl.ds(s * RPT + part * K, K)], bufa)
    pltpu.sync_copy(bufa, out_hbm.at[pl.ds(c * NP + s * RPT + part * K, K)])


_spmm128 = functools.partial(
    pl.kernel,
    out_type=jax.ShapeDtypeStruct((NC * NP, D_HID), jnp.float32),
    mesh=_mesh,
    scratch_types=[
        pltpu.VMEM((HCH, K), jnp.int32),
        pltpu.VMEM((HCH, K), jnp.int32),
        pltpu.VMEM((K, D_HID), jnp.float32),
        pltpu.VMEM((K, D_HID), jnp.float32),
        pltpu.SemaphoreType.DMA,
        pltpu.SemaphoreType.DMA,
        pltpu.VMEM_SHARED((NP, D_HID), jnp.float32),
    ],
)(_spmm_body)


_BR = 512


def _dis_body(d0_ref, d1_ref, mask_ref, out_ref):
  deg = d0_ref[...] + d1_ref[...] + 1.0
  out_ref[...] = mask_ref[...] * lax.rsqrt(deg)


def _dis_kernel(d0, d1, mask):
  return pl.pallas_call(
      _dis_body,
      out_shape=jax.ShapeDtypeStruct((NP, 1), jnp.float32),
  )(d0, d1, mask)


def _pre_body(a_ref, w_ref, dis_ref, out_ref):
  h = jnp.dot(a_ref[...], w_ref[...], preferred_element_type=jnp.float32)
  out_ref[...] = h * dis_ref[...]


def _pre_kernel(a, w, dis):
  din, dout = w.shape
  return pl.pallas_call(
      _pre_body,
      grid=(NP // _BR,),
      in_specs=[
          pl.BlockSpec((_BR, din), lambda i: (i, 0)),
          pl.BlockSpec((din, dout), lambda i: (0, 0)),
          pl.BlockSpec((_BR, 1), lambda i: (i, 0)),
      ],
      out_specs=pl.BlockSpec((_BR, dout), lambda i: (i, 0)),
      out_shape=jax.ShapeDtypeStruct((NP, dout), jnp.float32),
  )(a, w, dis)


def _mid_body(z0_ref, z1_ref, xp_ref, dis_ref, b_ref, w_ref, out_ref):
  h = dis_ref[...] * (z0_ref[...] + z1_ref[...] + xp_ref[...]) + b_ref[...]
  a = jnp.maximum(h, 0.0)
  out_ref[...] = (
      jnp.dot(a, w_ref[...], preferred_element_type=jnp.float32)
      * dis_ref[...])


def _mid_kernel(z0, z1, xp, dis, b, w):
  din, dout = w.shape
  return pl.pallas_call(
      _mid_body,
      grid=(NP // _BR,),
      in_specs=[
          pl.BlockSpec((_BR, din), lambda i: (i, 0)),
          pl.BlockSpec((_BR, din), lambda i: (i, 0)),
          pl.BlockSpec((_BR, din), lambda i: (i, 0)),
          pl.BlockSpec((_BR, 1), lambda i: (i, 0)),
          pl.BlockSpec((1, din), lambda i: (0, 0)),
          pl.BlockSpec((din, dout), lambda i: (0, 0)),
      ],
      out_specs=pl.BlockSpec((_BR, dout), lambda i: (i, 0)),
      out_shape=jax.ShapeDtypeStruct((NP, dout), jnp.float32),
  )(z0, z1, xp, dis, b, w)


def _final_body(z0_ref, z1_ref, xp_ref, dis_ref, b_ref, out_ref):
  h = dis_ref[...] * (z0_ref[...] + z1_ref[...] + xp_ref[...]) + b_ref[...]
  nrm = jnp.sqrt(jnp.sum(h * h, axis=1, keepdims=True))
  out_ref[...] = h / jnp.maximum(nrm, 1e-12)


def _final_kernel(z0, z1, xp, dis, b):
  d = b.shape[1]
  return pl.pallas_call(
      _final_body,
      grid=(NP // _BR,),
      in_specs=[
          pl.BlockSpec((_BR, d), lambda i: (i, 0)),
          pl.BlockSpec((_BR, d), lambda i: (i, 0)),
          pl.BlockSpec((_BR, d), lambda i: (i, 0)),
          pl.BlockSpec((_BR, 1), lambda i: (i, 0)),
          pl.BlockSpec((1, d), lambda i: (0, 0)),
      ],
      out_specs=pl.BlockSpec((_BR, d), lambda i: (i, 0)),
      out_shape=jax.ShapeDtypeStruct((NP, d), jnp.float32),
  )(z0, z1, xp, dis, b)


def kernel(x, edge_index, W1, b1, W2, b2, W3, b3):
  ei = edge_index.astype(jnp.int32)
  pad = jnp.full((EP - N_EDGES,), N_NODES, jnp.int32)
  srcp = jnp.concatenate([ei[0], pad]).reshape(NW, CH, K)
  dstp = jnp.concatenate([ei[1], pad]).reshape(NW, CH, K)

  xpad = jnp.pad(x, ((0, NP - N_NODES), (0, 0)))
  mask = (jnp.arange(NP) < N_NODES).astype(jnp.float32).reshape(NP, 1)

  degp = _deg_kernel(dstp)
  dis = _dis_kernel(degp[:NP, :1], degp[NP:, :1], mask)

  xp1 = _pre_kernel(xpad, W1, dis)
  zz = _spmm128(xp1, srcp, dstp)
  xp2 = _mid_kernel(zz[:NP], zz[NP:], xp1, dis, b1.reshape(1, -1), W2)
  zz = _spmm128(xp2, srcp, dstp)
  W3p = jnp.pad(W3, ((0, 0), (0, D_HID - D_EMB)))
  xp3 = _mid_kernel(zz[:NP], zz[NP:], xp2, dis, b2.reshape(1, -1), W3p)
  zz = _spmm128(xp3, srcp, dstp)
  emb = _final_kernel(zz[:NP, :D_EMB], zz[NP:, :D_EMB], xp3[:, :D_EMB],
                      dis, b3.reshape(1, -1))
  return emb[:N_NODES]
